# Initial kernel scaffold; baseline (speedup 1.0000x reference)
import jax
import jax.numpy as jnp
from jax.experimental import pallas as pl


def kernel(x, edge_index, W1, W2, b):
    raise NotImplementedError("write your pallas kernel here")


# speedup vs baseline: 24.3839x; 24.3839x over previous
import functools

import jax
import jax.numpy as jnp
from jax import lax
from jax.experimental import pallas as pl
from jax.experimental.pallas import tpu as pltpu
from jax.experimental.pallas import tpu_sc as plsc

NC, NS, LANES = 2, 16, 16
BS = 64


def _sc_mesh():
    return plsc.VectorSubcoreMesh(
        core_axis_name="c", subcore_axis_name="s",
        num_cores=NC, num_subcores=NS)


_SC_PARAMS = pltpu.CompilerParams(use_tc_tiling_on_sc=False)


def _row_split(n):
    per = -(-(n // NS) // 8) * 8
    last = n - per * (NS - 1)
    assert last > 0 and last % 8 == 0
    return per, last


def _tile_copy(src, dst, sid, per, last):
    @pl.when(sid < NS - 1)
    def _():
        off = pl.multiple_of(sid * per, 8)
        pltpu.sync_copy(src.at[pl.ds(off, per)], dst.at[pl.ds(off, per)])

    @pl.when(sid == NS - 1)
    def _():
        off = (NS - 1) * per
        pltpu.sync_copy(src.at[pl.ds(off, last)], dst.at[pl.ds(off, last)])


def _batch_split(e):
    nbt = e // BS
    assert nbt * BS == e
    q, rem = divmod(nbt, NC * NS)
    return nbt, q, rem


def _make_sc_deg(n, e):
    nbt, q, rem = _batch_split(e)
    per, last = _row_split(n)

    @functools.partial(
        pl.kernel,
        out_type=jax.ShapeDtypeStruct((NC, n, LANES), jnp.float32),
        mesh=_sc_mesh(),
        scratch_types=[
            pltpu.VMEM((q + 1, BS), jnp.int32),
            pltpu.VMEM((BS, LANES), jnp.float32),
            pltpu.SemaphoreType.DMA,
            pltpu.VMEM_SHARED((n, LANES), jnp.float32),
        ],
        compiler_params=_SC_PARAMS,
    )
    def deg_kernel(col2d_hbm, zeros_hbm, out_hbm, idx_all, ones_v, sem, acc):
        cid = lax.axis_index("c")
        sid = lax.axis_index("s")
        wid = cid * NS + sid
        extra = wid < rem
        r0 = wid * q + jnp.minimum(wid, rem)
        nb = q + extra.astype(jnp.int32)

        def fill(i, _):
            ones_v[i, :] = jnp.full((LANES,), 1.0, jnp.float32)
            return 0
        lax.fori_loop(0, BS, fill, 0)

        pltpu.sync_copy(col2d_hbm.at[pl.ds(r0, q)], idx_all.at[pl.ds(0, q)])

        @pl.when(extra)
        def _():
            pltpu.sync_copy(col2d_hbm.at[pl.ds(r0 + q, 1)],
                            idx_all.at[pl.ds(q, 1)])

        _tile_copy(zeros_hbm, acc, sid, per, last)
        plsc.subcore_barrier()

        def body(bi, _):
            pltpu.async_copy(ones_v, acc.at[idx_all.at[bi]], sem, add=True)
            return 0
        lax.fori_loop(0, nb, body, 0)

        def drain(bi, _):
            pltpu.make_async_copy(ones_v, acc.at[idx_all.at[0]], sem).wait()
            return 0
        lax.fori_loop(0, nb, drain, 0)

        plsc.subcore_barrier()
        _tile_copy(acc, out_hbm.at[cid], sid, per, last)

    return deg_kernel


def _make_sc_spmm(n, e, w):
    nbt, q, rem = _batch_split(e)
    per, last = _row_split(n)
    NBUF, NIDX = (5 if w <= 128 else 4), 8

    @functools.partial(
        pl.kernel,
        out_type=jax.ShapeDtypeStruct((NC, n, w), jnp.float32),
        mesh=_sc_mesh(),
        scratch_types=[
            pltpu.VMEM((NIDX, BS), jnp.int32),
            pltpu.VMEM((NIDX, BS), jnp.int32),
            pltpu.VMEM((NBUF, BS, w), jnp.float32),
            pltpu.SemaphoreType.DMA,
            pltpu.SemaphoreType.DMA,
            pltpu.SemaphoreType.DMA,
            pltpu.VMEM_SHARED((n, w), jnp.float32),
        ],
        compiler_params=_SC_PARAMS,
    )
    def spmm_kernel(row2d_hbm, col2d_hbm, tab_hbm, zeros_hbm, out_hbm,
                    idxr, idxc, bufs, semi, semg, sems, acc):
        cid = lax.axis_index("c")
        sid = lax.axis_index("s")
        wid = cid * NS + sid
        extra = wid < rem
        r0 = wid * q + jnp.minimum(wid, rem)
        nb = q + extra.astype(jnp.int32)

        def idx_start(bi):
            s = bi % NIDX
            pltpu.async_copy(row2d_hbm.at[pl.ds(r0 + bi, 1)],
                             idxr.at[pl.ds(s, 1)], semi)
            pltpu.async_copy(col2d_hbm.at[pl.ds(r0 + bi, 1)],
                             idxc.at[pl.ds(s, 1)], semi)

        def idx_wait():
            pltpu.make_async_copy(row2d_hbm.at[pl.ds(r0, 1)],
                                  idxr.at[pl.ds(0, 1)], semi).wait()
            pltpu.make_async_copy(col2d_hbm.at[pl.ds(r0, 1)],
                                  idxc.at[pl.ds(0, 1)], semi).wait()

        def g_start(bi):
            pltpu.async_copy(tab_hbm.at[idxr.at[bi % NIDX]],
                             bufs.at[bi % NBUF], semg)

        def g_wait():
            pltpu.make_async_copy(tab_hbm.at[idxr.at[0]],
                                  bufs.at[0], semg).wait()

        def s_start(bi):
            pltpu.async_copy(bufs.at[bi % NBUF],
                             acc.at[idxc.at[bi % NIDX]], sems,
                             add=True)

        def s_drain():
            pltpu.make_async_copy(bufs.at[0], acc.at[idxc.at[0]],
                                  sems).wait()

        _tile_copy(zeros_hbm, acc, sid, per, last)
        plsc.subcore_barrier()

        for bi in range(NBUF):
            idx_start(bi)
        for bi in range(NBUF - 1):
            idx_wait()
            g_start(bi)

        def body(b, _):
            @pl.when(b >= 1)
            def _():
                s_drain()

            @pl.when(b + NBUF < nb)
            def _():
                idx_start(b + NBUF)

            @pl.when(b + NBUF - 1 < nb)
            def _():
                idx_wait()
                g_start(b + NBUF - 1)
            g_wait()
            s_start(b)
            return 0
        lax.fori_loop(0, nb, body, 0)
        s_drain()

        plsc.subcore_barrier()
        _tile_copy(acc, out_hbm.at[cid], sid, per, last)

    return spmm_kernel


def _gelu(v):
    return 0.5 * v * (1.0 + lax.erf(v * (2.0 ** -0.5)))


def _tc0_body(x_ref, w_ref, z_ref):
    z_ref[...] = lax.dot_general(x_ref[...], w_ref[...],
                                 (((1,), (1,)), ((), ())),
                                 preferred_element_type=jnp.float32)


def _tc1_body(x_ref, w_ref, degp_ref, aug_ref):
    z3 = lax.dot_general(x_ref[...], w_ref[...], (((1,), (1,)), ((), ())),
                         preferred_element_type=jnp.float32)
    deg = degp_ref[0, :, 0] + degp_ref[1, :, 0]
    inv = 1.0 / deg
    aug_ref[...] = jnp.concatenate(
        [z3 * inv[:, None],
         jnp.broadcast_to(inv[:, None], (z3.shape[0], LANES))], axis=1)


def _tc2_body(h, z12_ref, b0_ref, aggp_ref, degp_ref, w_ref, b_ref,
              o1_ref, bs1_ref):
    agg = aggp_ref[0, :, :h] + aggp_ref[1, :, :h]
    c = aggp_ref[0, :, h] + aggp_ref[1, :, h]
    x1 = _gelu(z12_ref[:, :h] + b0_ref[...] - agg
               - c[:, None] * z12_ref[:, h:])
    z = lax.dot_general(x1, w_ref[...], (((1,), (1,)), ((), ())),
                        preferred_element_type=jnp.float32)
    deg = degp_ref[0, :, 0] + degp_ref[1, :, 0]
    inv = 1.0 / deg
    o1_ref[...] = z[:, :h] + b_ref[...] - c[:, None] * z[:, h:2 * h]
    bs1_ref[...] = z[:, 2 * h:] * inv[:, None]


def _tc3_body(o1_ref, aggp_ref, out_ref):
    out_ref[...] = _gelu(o1_ref[...] - aggp_ref[0] - aggp_ref[1])


def kernel(x, edge_index, W1, W2, b):
    n, h = x.shape
    e = edge_index.shape[1]
    row2d = edge_index[0].reshape(e // BS, BS)
    col2d = edge_index[1].reshape(e // BS, BS)

    wcat12 = jnp.concatenate([W1[0], W2[0][:, :h]], axis=0)
    wb0 = W2[0][:, h:]
    wcat1 = jnp.concatenate([W1[1], W2[1][:, :h], W2[1][:, h:]], axis=0)
    b0, b1 = b[0][None, :], b[1][None, :]

    z16 = jnp.zeros((n, LANES), jnp.float32)
    zaug = jnp.zeros((n, h + LANES), jnp.float32)
    z128 = jnp.zeros((n, h), jnp.float32)

    degp = _make_sc_deg(n, e)(col2d, z16)

    r = 1000
    grid = (n // r,)
    f32 = jnp.float32

    z12 = pl.pallas_call(
        _tc0_body,
        grid=grid,
        in_specs=[
            pl.BlockSpec((r, h), lambda i: (i, 0)),
            pl.BlockSpec((2 * h, h), lambda i: (0, 0)),
        ],
        out_specs=pl.BlockSpec((r, 2 * h), lambda i: (i, 0)),
        out_shape=jax.ShapeDtypeStruct((n, 2 * h), f32),
    )(x, wcat12)

    aug = pl.pallas_call(
        _tc1_body,
        grid=grid,
        in_specs=[
            pl.BlockSpec((r, h), lambda i: (i, 0)),
            pl.BlockSpec((h, h), lambda i: (0, 0)),
            pl.BlockSpec((NC, r, LANES), lambda i: (0, i, 0)),
        ],
        out_specs=pl.BlockSpec((r, h + LANES), lambda i: (i, 0)),
        out_shape=jax.ShapeDtypeStruct((n, h + LANES), f32),
    )(x, wb0, degp)

    aggp0 = _make_sc_spmm(n, e, h + LANES)(row2d, col2d, aug, zaug)

    o1, bs1 = pl.pallas_call(
        functools.partial(_tc2_body, h),
        grid=grid,
        in_specs=[
            pl.BlockSpec((r, 2 * h), lambda i: (i, 0)),
            pl.BlockSpec((1, h), lambda i: (0, 0)),
            pl.BlockSpec((NC, r, h + LANES), lambda i: (0, i, 0)),
            pl.BlockSpec((NC, r, LANES), lambda i: (0, i, 0)),
            pl.BlockSpec((3 * h, h), lambda i: (0, 0)),
            pl.BlockSpec((1, h), lambda i: (0, 0)),
        ],
        out_specs=[
            pl.BlockSpec((r, h), lambda i: (i, 0)),
            pl.BlockSpec((r, h), lambda i: (i, 0)),
        ],
        out_shape=[
            jax.ShapeDtypeStruct((n, h), f32),
            jax.ShapeDtypeStruct((n, h), f32),
        ],
    )(z12, b0, aggp0, degp, wcat1, b1)

    aggp1 = _make_sc_spmm(n, e, h)(row2d, col2d, bs1, z128)

    out = pl.pallas_call(
        _tc3_body,
        grid=grid,
        in_specs=[
            pl.BlockSpec((r, h), lambda i: (i, 0)),
            pl.BlockSpec((NC, r, h), lambda i: (0, i, 0)),
        ],
        out_specs=pl.BlockSpec((r, h), lambda i: (i, 0)),
        out_shape=jax.ShapeDtypeStruct((n, h), f32),
    )(o1, aggp1)

    return out


# speedup vs baseline: 24.6328x; 1.0102x over previous
import functools

import jax
import jax.numpy as jnp
from jax import lax
from jax.experimental import pallas as pl
from jax.experimental.pallas import tpu as pltpu
from jax.experimental.pallas import tpu_sc as plsc

NC, NS, LANES = 2, 16, 16
BS = 64


def _sc_mesh():
    return plsc.VectorSubcoreMesh(
        core_axis_name="c", subcore_axis_name="s",
        num_cores=NC, num_subcores=NS)


_SC_PARAMS = pltpu.CompilerParams(use_tc_tiling_on_sc=False)


def _row_split(n):
    per = -(-(n // NS) // 8) * 8
    last = n - per * (NS - 1)
    assert last > 0 and last % 8 == 0
    return per, last


def _tile_copy(src, dst, sid, per, last):
    @pl.when(sid < NS - 1)
    def _():
        off = pl.multiple_of(sid * per, 8)
        pltpu.sync_copy(src.at[pl.ds(off, per)], dst.at[pl.ds(off, per)])

    @pl.when(sid == NS - 1)
    def _():
        off = (NS - 1) * per
        pltpu.sync_copy(src.at[pl.ds(off, last)], dst.at[pl.ds(off, last)])


def _batch_split(e):
    nbt = e // BS
    assert nbt * BS == e
    q, rem = divmod(nbt, NC * NS)
    return nbt, q, rem


def _make_sc_deg(n, e):
    nbt, q, rem = _batch_split(e)
    per, last = _row_split(n)

    @functools.partial(
        pl.kernel,
        out_type=jax.ShapeDtypeStruct((NC, n, LANES), jnp.float32),
        mesh=_sc_mesh(),
        scratch_types=[
            pltpu.VMEM((q + 1, BS), jnp.int32),
            pltpu.VMEM((BS, LANES), jnp.float32),
            pltpu.SemaphoreType.DMA,
            pltpu.VMEM_SHARED((n, LANES), jnp.float32),
        ],
        compiler_params=_SC_PARAMS,
    )
    def deg_kernel(col2d_hbm, zeros_hbm, out_hbm, idx_all, ones_v, sem, acc):
        cid = lax.axis_index("c")
        sid = lax.axis_index("s")
        wid = cid * NS + sid
        extra = wid < rem
        r0 = wid * q + jnp.minimum(wid, rem)
        nb = q + extra.astype(jnp.int32)

        def fill(i, _):
            ones_v[i, :] = jnp.full((LANES,), 1.0, jnp.float32)
            return 0
        lax.fori_loop(0, BS, fill, 0)

        pltpu.sync_copy(col2d_hbm.at[pl.ds(r0, q)], idx_all.at[pl.ds(0, q)])

        @pl.when(extra)
        def _():
            pltpu.sync_copy(col2d_hbm.at[pl.ds(r0 + q, 1)],
                            idx_all.at[pl.ds(q, 1)])

        _tile_copy(zeros_hbm, acc, sid, per, last)
        plsc.subcore_barrier()

        def body(bi, _):
            pltpu.async_copy(ones_v, acc.at[idx_all.at[bi]], sem, add=True)
            return 0
        lax.fori_loop(0, nb, body, 0)

        def drain(bi, _):
            pltpu.make_async_copy(ones_v, acc.at[idx_all.at[0]], sem).wait()
            return 0
        lax.fori_loop(0, nb, drain, 0)

        plsc.subcore_barrier()
        _tile_copy(acc, out_hbm.at[cid], sid, per, last)

    return deg_kernel


def _make_sc_spmm(n, e, w):
    nbt, q, rem = _batch_split(e)
    per, last = _row_split(n)
    NBUF, NIDX = (5 if w <= 128 else 4), 8

    @functools.partial(
        pl.kernel,
        out_type=jax.ShapeDtypeStruct((NC, n, w), jnp.float32),
        mesh=_sc_mesh(),
        scratch_types=[
            pltpu.VMEM((NIDX, BS), jnp.int32),
            pltpu.VMEM((NIDX, BS), jnp.int32),
            pltpu.VMEM((NBUF, BS, w), jnp.float32),
            pltpu.SemaphoreType.DMA,
            pltpu.SemaphoreType.DMA,
            pltpu.SemaphoreType.DMA,
            pltpu.VMEM_SHARED((n, w), jnp.float32),
        ],
        compiler_params=_SC_PARAMS,
    )
    def spmm_kernel(row2d_hbm, col2d_hbm, tab_hbm, zeros_hbm, out_hbm,
                    idxr, idxc, bufs, semi, semg, sems, acc):
        cid = lax.axis_index("c")
        sid = lax.axis_index("s")
        wid = cid * NS + sid
        extra = wid < rem
        r0 = wid * q + jnp.minimum(wid, rem)
        nb = q + extra.astype(jnp.int32)

        def idx_start(bi):
            s = bi % NIDX
            pltpu.async_copy(row2d_hbm.at[pl.ds(r0 + bi, 1)],
                             idxr.at[pl.ds(s, 1)], semi)
            pltpu.async_copy(col2d_hbm.at[pl.ds(r0 + bi, 1)],
                             idxc.at[pl.ds(s, 1)], semi)

        def idx_wait():
            pltpu.make_async_copy(row2d_hbm.at[pl.ds(r0, 1)],
                                  idxr.at[pl.ds(0, 1)], semi).wait()
            pltpu.make_async_copy(col2d_hbm.at[pl.ds(r0, 1)],
                                  idxc.at[pl.ds(0, 1)], semi).wait()

        def g_start(bi):
            pltpu.async_copy(tab_hbm.at[idxr.at[bi % NIDX]],
                             bufs.at[bi % NBUF], semg)

        def g_wait():
            pltpu.make_async_copy(tab_hbm.at[idxr.at[0]],
                                  bufs.at[0], semg).wait()

        def s_start(bi):
            pltpu.async_copy(bufs.at[bi % NBUF],
                             acc.at[idxc.at[bi % NIDX]], sems,
                             add=True)

        def s_drain():
            pltpu.make_async_copy(bufs.at[0], acc.at[idxc.at[0]],
                                  sems).wait()

        _tile_copy(zeros_hbm, acc, sid, per, last)
        plsc.subcore_barrier()

        for bi in range(NBUF):
            idx_start(bi)
        for bi in range(NBUF - 1):
            idx_wait()
            g_start(bi)

        def body(b, _):
            @pl.when(b >= 1)
            def _():
                s_drain()

            @pl.when(b + NBUF < nb)
            def _():
                idx_start(b + NBUF)

            @pl.when(b + NBUF - 1 < nb)
            def _():
                idx_wait()
                g_start(b + NBUF - 1)
            g_wait()
            s_start(b)
            return 0
        lax.fori_loop(0, nb, body, 0)
        s_drain()

        plsc.subcore_barrier()
        _tile_copy(acc, out_hbm.at[cid], sid, per, last)

    return spmm_kernel


def _gelu(v):
    return 0.5 * v * (1.0 + lax.erf(v * (2.0 ** -0.5)))


def _tc0_body(x_ref, w_ref, z_ref):
    z_ref[...] = lax.dot_general(x_ref[...], w_ref[...],
                                 (((1,), (1,)), ((), ())),
                                 preferred_element_type=jnp.float32)


def _tc1_body(x_ref, w_ref, degp_ref, aug_ref):
    z3 = lax.dot_general(x_ref[...], w_ref[...], (((1,), (1,)), ((), ())),
                         preferred_element_type=jnp.float32)
    deg = degp_ref[0, :, 0] + degp_ref[1, :, 0]
    inv = 1.0 / deg
    aug_ref[...] = jnp.concatenate(
        [z3 * inv[:, None],
         jnp.broadcast_to(inv[:, None], (z3.shape[0], LANES))], axis=1)


def _tc2_body(h, z12_ref, b0_ref, aggp_ref, degp_ref, w_ref, b_ref,
              o1_ref, bs1_ref):
    agg = aggp_ref[0, :, :h] + aggp_ref[1, :, :h]
    c = aggp_ref[0, :, h] + aggp_ref[1, :, h]
    x1 = _gelu(z12_ref[:, :h] + b0_ref[...] - agg
               - c[:, None] * z12_ref[:, h:])
    z = lax.dot_general(x1, w_ref[...], (((1,), (1,)), ((), ())),
                        preferred_element_type=jnp.float32)
    deg = degp_ref[0, :, 0] + degp_ref[1, :, 0]
    inv = 1.0 / deg
    o1_ref[...] = 0.5 * (z[:, :h] + b_ref[...] - c[:, None] * z[:, h:2 * h])
    bs1_ref[...] = -(z[:, 2 * h:] * inv[:, None])


def _tc3_body(aggp_ref, out_ref):
    out_ref[...] = _gelu(aggp_ref[0] + aggp_ref[1])


def kernel(x, edge_index, W1, W2, b):
    n, h = x.shape
    e = edge_index.shape[1]
    row2d = edge_index[0].reshape(e // BS, BS)
    col2d = edge_index[1].reshape(e // BS, BS)

    wcat12 = jnp.concatenate([W1[0], W2[0][:, :h]], axis=0)
    wb0 = W2[0][:, h:]
    wcat1 = jnp.concatenate([W1[1], W2[1][:, :h], W2[1][:, h:]], axis=0)
    b0, b1 = b[0][None, :], b[1][None, :]

    z16 = jnp.zeros((n, LANES), jnp.float32)
    zaug = jnp.zeros((n, h + LANES), jnp.float32)

    degp = _make_sc_deg(n, e)(col2d, z16)

    r = 1000
    grid = (n // r,)
    f32 = jnp.float32

    z12 = pl.pallas_call(
        _tc0_body,
        grid=grid,
        in_specs=[
            pl.BlockSpec((r, h), lambda i: (i, 0)),
            pl.BlockSpec((2 * h, h), lambda i: (0, 0)),
        ],
        out_specs=pl.BlockSpec((r, 2 * h), lambda i: (i, 0)),
        out_shape=jax.ShapeDtypeStruct((n, 2 * h), f32),
    )(x, wcat12)

    aug = pl.pallas_call(
        _tc1_body,
        grid=grid,
        in_specs=[
            pl.BlockSpec((r, h), lambda i: (i, 0)),
            pl.BlockSpec((h, h), lambda i: (0, 0)),
            pl.BlockSpec((NC, r, LANES), lambda i: (0, i, 0)),
        ],
        out_specs=pl.BlockSpec((r, h + LANES), lambda i: (i, 0)),
        out_shape=jax.ShapeDtypeStruct((n, h + LANES), f32),
    )(x, wb0, degp)

    aggp0 = _make_sc_spmm(n, e, h + LANES)(row2d, col2d, aug, zaug)

    o1, bs1 = pl.pallas_call(
        functools.partial(_tc2_body, h),
        grid=grid,
        in_specs=[
            pl.BlockSpec((r, 2 * h), lambda i: (i, 0)),
            pl.BlockSpec((1, h), lambda i: (0, 0)),
            pl.BlockSpec((NC, r, h + LANES), lambda i: (0, i, 0)),
            pl.BlockSpec((NC, r, LANES), lambda i: (0, i, 0)),
            pl.BlockSpec((3 * h, h), lambda i: (0, 0)),
            pl.BlockSpec((1, h), lambda i: (0, 0)),
        ],
        out_specs=[
            pl.BlockSpec((r, h), lambda i: (i, 0)),
            pl.BlockSpec((r, h), lambda i: (i, 0)),
        ],
        out_shape=[
            jax.ShapeDtypeStruct((n, h), f32),
            jax.ShapeDtypeStruct((n, h), f32),
        ],
    )(z12, b0, aggp0, degp, wcat1, b1)

    aggp1 = _make_sc_spmm(n, e, h)(row2d, col2d, bs1, o1)

    out = pl.pallas_call(
        _tc3_body,
        grid=grid,
        in_specs=[
            pl.BlockSpec((NC, r, h), lambda i: (0, i, 0)),
        ],
        out_specs=pl.BlockSpec((r, h), lambda i: (i, 0)),
        out_shape=jax.ShapeDtypeStruct((n, h), f32),
    )(aggp1)

    return out


# speedup vs baseline: 24.7326x; 1.0041x over previous
import functools

import jax
import jax.numpy as jnp
from jax import lax
from jax.experimental import pallas as pl
from jax.experimental.pallas import tpu as pltpu
from jax.experimental.pallas import tpu_sc as plsc

NC, NS, LANES = 2, 16, 16
BS = 64


def _sc_mesh():
    return plsc.VectorSubcoreMesh(
        core_axis_name="c", subcore_axis_name="s",
        num_cores=NC, num_subcores=NS)


_SC_PARAMS = pltpu.CompilerParams(use_tc_tiling_on_sc=False)


def _row_split(n):
    per = -(-(n // NS) // 8) * 8
    last = n - per * (NS - 1)
    assert last > 0 and last % 8 == 0
    return per, last


def _tile_copy(src, dst, sid, per, last):
    @pl.when(sid < NS - 1)
    def _():
        off = pl.multiple_of(sid * per, 8)
        pltpu.sync_copy(src.at[pl.ds(off, per)], dst.at[pl.ds(off, per)])

    @pl.when(sid == NS - 1)
    def _():
        off = (NS - 1) * per
        pltpu.sync_copy(src.at[pl.ds(off, last)], dst.at[pl.ds(off, last)])


def _batch_split(e, bs=BS):
    nbt = e // bs
    assert nbt * bs == e
    q, rem = divmod(nbt, NC * NS)
    return nbt, q, rem


def _make_sc_deg(n, e):
    nbt, q, rem = _batch_split(e)
    per, last = _row_split(n)

    @functools.partial(
        pl.kernel,
        out_type=jax.ShapeDtypeStruct((NC, n, LANES), jnp.float32),
        mesh=_sc_mesh(),
        scratch_types=[
            pltpu.VMEM((q + 1, BS), jnp.int32),
            pltpu.VMEM((BS, LANES), jnp.float32),
            pltpu.SemaphoreType.DMA,
            pltpu.VMEM_SHARED((n, LANES), jnp.float32),
        ],
        compiler_params=_SC_PARAMS,
    )
    def deg_kernel(col2d_hbm, zeros_hbm, out_hbm, idx_all, ones_v, sem, acc):
        cid = lax.axis_index("c")
        sid = lax.axis_index("s")
        wid = cid * NS + sid
        extra = wid < rem
        r0 = wid * q + jnp.minimum(wid, rem)
        nb = q + extra.astype(jnp.int32)

        def fill(i, _):
            ones_v[i, :] = jnp.full((LANES,), 1.0, jnp.float32)
            return 0
        lax.fori_loop(0, BS, fill, 0)

        pltpu.sync_copy(col2d_hbm.at[pl.ds(r0, q)], idx_all.at[pl.ds(0, q)])

        @pl.when(extra)
        def _():
            pltpu.sync_copy(col2d_hbm.at[pl.ds(r0 + q, 1)],
                            idx_all.at[pl.ds(q, 1)])

        _tile_copy(zeros_hbm, acc, sid, per, last)
        plsc.subcore_barrier()

        def body(bi, _):
            pltpu.async_copy(ones_v, acc.at[idx_all.at[bi]], sem, add=True)
            return 0
        lax.fori_loop(0, nb, body, 0)

        def drain(bi, _):
            pltpu.make_async_copy(ones_v, acc.at[idx_all.at[0]], sem).wait()
            return 0
        lax.fori_loop(0, nb, drain, 0)

        plsc.subcore_barrier()
        _tile_copy(acc, out_hbm.at[cid], sid, per, last)

    return deg_kernel


def _make_sc_spmm(n, e, w, bs=BS, nbuf=4, nidx=8):
    nbt, q, rem = _batch_split(e, bs)
    per, last = _row_split(n)
    NBUF, NIDX = nbuf, nidx

    @functools.partial(
        pl.kernel,
        out_type=jax.ShapeDtypeStruct((NC, n, w), jnp.float32),
        mesh=_sc_mesh(),
        scratch_types=[
            pltpu.VMEM((NIDX, bs), jnp.int32),
            pltpu.VMEM((NIDX, bs), jnp.int32),
            pltpu.VMEM((NBUF, bs, w), jnp.float32),
            pltpu.SemaphoreType.DMA,
            pltpu.SemaphoreType.DMA,
            pltpu.SemaphoreType.DMA,
            pltpu.VMEM_SHARED((n, w), jnp.float32),
        ],
        compiler_params=_SC_PARAMS,
    )
    def spmm_kernel(row2d_hbm, col2d_hbm, tab_hbm, zeros_hbm, out_hbm,
                    idxr, idxc, bufs, semi, semg, sems, acc):
        cid = lax.axis_index("c")
        sid = lax.axis_index("s")
        wid = cid * NS + sid
        extra = wid < rem
        r0 = wid * q + jnp.minimum(wid, rem)
        nb = q + extra.astype(jnp.int32)

        def idx_start(bi):
            s = bi % NIDX
            pltpu.async_copy(row2d_hbm.at[pl.ds(r0 + bi, 1)],
                             idxr.at[pl.ds(s, 1)], semi)
            pltpu.async_copy(col2d_hbm.at[pl.ds(r0 + bi, 1)],
                             idxc.at[pl.ds(s, 1)], semi)

        def idx_wait():
            pltpu.make_async_copy(row2d_hbm.at[pl.ds(r0, 1)],
                                  idxr.at[pl.ds(0, 1)], semi).wait()
            pltpu.make_async_copy(col2d_hbm.at[pl.ds(r0, 1)],
                                  idxc.at[pl.ds(0, 1)], semi).wait()

        def g_start(bi):
            pltpu.async_copy(tab_hbm.at[idxr.at[bi % NIDX]],
                             bufs.at[bi % NBUF], semg)

        def g_wait():
            pltpu.make_async_copy(tab_hbm.at[idxr.at[0]],
                                  bufs.at[0], semg).wait()

        def s_start(bi):
            pltpu.async_copy(bufs.at[bi % NBUF],
                             acc.at[idxc.at[bi % NIDX]], sems,
                             add=True)

        def s_drain():
            pltpu.make_async_copy(bufs.at[0], acc.at[idxc.at[0]],
                                  sems).wait()

        _tile_copy(zeros_hbm, acc, sid, per, last)
        plsc.subcore_barrier()

        for bi in range(NBUF):
            idx_start(bi)
        for bi in range(NBUF - 1):
            idx_wait()
            g_start(bi)

        def body(b, _):
            @pl.when(b >= 1)
            def _():
                s_drain()

            @pl.when(b + NBUF < nb)
            def _():
                idx_start(b + NBUF)

            @pl.when(b + NBUF - 1 < nb)
            def _():
                idx_wait()
                g_start(b + NBUF - 1)
            g_wait()
            s_start(b)
            return 0
        lax.fori_loop(0, nb, body, 0)
        s_drain()

        plsc.subcore_barrier()
        _tile_copy(acc, out_hbm.at[cid], sid, per, last)

    return spmm_kernel


def _gelu(v):
    return 0.5 * v * (1.0 + lax.erf(v * (2.0 ** -0.5)))


def _tc0_body(x_ref, w_ref, z_ref):
    z_ref[...] = lax.dot_general(x_ref[...], w_ref[...],
                                 (((1,), (1,)), ((), ())),
                                 preferred_element_type=jnp.float32)


def _tc1_body(x_ref, w_ref, degp_ref, aug_ref):
    z3 = lax.dot_general(x_ref[...], w_ref[...], (((1,), (1,)), ((), ())),
                         preferred_element_type=jnp.float32)
    deg = degp_ref[0, :, 0] + degp_ref[1, :, 0]
    inv = 1.0 / deg
    aug_ref[...] = jnp.concatenate(
        [z3 * inv[:, None],
         jnp.broadcast_to(inv[:, None], (z3.shape[0], LANES))], axis=1)


def _tc2_body(h, z12_ref, b0_ref, aggp_ref, degp_ref, w_ref, b_ref,
              o1_ref, bs1_ref):
    agg = aggp_ref[0, :, :h] + aggp_ref[1, :, :h]
    c = aggp_ref[0, :, h] + aggp_ref[1, :, h]
    x1 = _gelu(z12_ref[:, :h] + b0_ref[...] - agg
               - c[:, None] * z12_ref[:, h:])
    z = lax.dot_general(x1, w_ref[...], (((1,), (1,)), ((), ())),
                        preferred_element_type=jnp.float32)
    deg = degp_ref[0, :, 0] + degp_ref[1, :, 0]
    inv = 1.0 / deg
    o1_ref[...] = 0.5 * (z[:, :h] + b_ref[...] - c[:, None] * z[:, h:2 * h])
    bs1_ref[...] = -(z[:, 2 * h:] * inv[:, None])


def _tc3_body(aggp_ref, out_ref):
    out_ref[...] = _gelu(aggp_ref[0] + aggp_ref[1])


def kernel(x, edge_index, W1, W2, b):
    n, h = x.shape
    e = edge_index.shape[1]
    row2d = edge_index[0].reshape(e // BS, BS)
    col2d = edge_index[1].reshape(e // BS, BS)

    wcat12 = jnp.concatenate([W1[0], W2[0][:, :h]], axis=0)
    wb0 = W2[0][:, h:]
    wcat1 = jnp.concatenate([W1[1], W2[1][:, :h], W2[1][:, h:]], axis=0)
    b0, b1 = b[0][None, :], b[1][None, :]

    z16 = jnp.zeros((n, LANES), jnp.float32)
    zaug = jnp.zeros((n, h + LANES), jnp.float32)

    degp = _make_sc_deg(n, e)(col2d, z16)

    r = 1000
    grid = (n // r,)
    f32 = jnp.float32

    z12 = pl.pallas_call(
        _tc0_body,
        grid=grid,
        in_specs=[
            pl.BlockSpec((r, h), lambda i: (i, 0)),
            pl.BlockSpec((2 * h, h), lambda i: (0, 0)),
        ],
        out_specs=pl.BlockSpec((r, 2 * h), lambda i: (i, 0)),
        out_shape=jax.ShapeDtypeStruct((n, 2 * h), f32),
    )(x, wcat12)

    aug = pl.pallas_call(
        _tc1_body,
        grid=grid,
        in_specs=[
            pl.BlockSpec((r, h), lambda i: (i, 0)),
            pl.BlockSpec((h, h), lambda i: (0, 0)),
            pl.BlockSpec((NC, r, LANES), lambda i: (0, i, 0)),
        ],
        out_specs=pl.BlockSpec((r, h + LANES), lambda i: (i, 0)),
        out_shape=jax.ShapeDtypeStruct((n, h + LANES), f32),
    )(x, wb0, degp)

    aggp0 = _make_sc_spmm(n, e, h + LANES)(row2d, col2d, aug, zaug)

    o1, bs1 = pl.pallas_call(
        functools.partial(_tc2_body, h),
        grid=grid,
        in_specs=[
            pl.BlockSpec((r, 2 * h), lambda i: (i, 0)),
            pl.BlockSpec((1, h), lambda i: (0, 0)),
            pl.BlockSpec((NC, r, h + LANES), lambda i: (0, i, 0)),
            pl.BlockSpec((NC, r, LANES), lambda i: (0, i, 0)),
            pl.BlockSpec((3 * h, h), lambda i: (0, 0)),
            pl.BlockSpec((1, h), lambda i: (0, 0)),
        ],
        out_specs=[
            pl.BlockSpec((r, h), lambda i: (i, 0)),
            pl.BlockSpec((r, h), lambda i: (i, 0)),
        ],
        out_shape=[
            jax.ShapeDtypeStruct((n, h), f32),
            jax.ShapeDtypeStruct((n, h), f32),
        ],
    )(z12, b0, aggp0, degp, wcat1, b1)

    row2dw = edge_index[0].reshape(e // 128, 128)
    col2dw = edge_index[1].reshape(e // 128, 128)
    aggp1 = _make_sc_spmm(n, e, h, bs=128, nbuf=3, nidx=4)(
        row2dw, col2dw, bs1, o1)

    out = pl.pallas_call(
        _tc3_body,
        grid=grid,
        in_specs=[
            pl.BlockSpec((NC, r, h), lambda i: (0, i, 0)),
        ],
        out_specs=pl.BlockSpec((r, h), lambda i: (i, 0)),
        out_shape=jax.ShapeDtypeStruct((n, h), f32),
    )(aggp1)

    return out
